# baseline (speedup 1.0000x reference)
import jax
import jax.numpy as jnp
from jax import lax
from jax.experimental import pallas as pl
from jax.experimental.pallas import tpu as pltpu
from jax.experimental.pallas import tpu_sc as plsc

_B = 100000
_C = 128
_N = 16384
_L = 16
_NC = 2
_NS = 16
_NW = _NC * _NS
_RPW = 3128
_RLAST = _B - (_NW - 1) * _RPW
_NG = _N // _L
_CHUNK = 128
_NFULL = _RPW // _CHUNK
_REM = _RPW - _NFULL * _CHUNK
_NFULL_LAST = _RLAST // _CHUNK
_REM_LAST = _RLAST - _NFULL_LAST * _CHUNK
_WPAD = ((_RPW + _L - 1) // _L) * _L


def _sc_body(inp_hbm, idx_hbm, out_hbm,
             idx_v, win_v, glist_v, plist_v, cbuf_v, chunk_v, gsem):
  wid = lax.axis_index("s") * _NC + lax.axis_index("c")
  base = wid * _RPW
  nrows_w = jnp.where(wid == _NW - 1, _RLAST, _RPW)
  iota = lax.iota(jnp.int32, _L)

  pltpu.sync_copy(idx_hbm, idx_v)

  def init_w(g, _):
    win_v[pl.ds(g * _L, _L)] = jnp.full((_L,), -1, jnp.int32)
    return 0
  lax.fori_loop(0, _WPAD // _L, init_w, 0)

  def zero_row(r, _):
    def zc(l, _):
      chunk_v[r, pl.ds(l * _L, _L)] = jnp.zeros((_L,), jnp.float32)
      return 0
    lax.fori_loop(0, _C // _L, zc, 0)
    return 0
  lax.fori_loop(0, _CHUNK, zero_row, 0)

  def scan(g, _):
    i16 = idx_v[pl.ds(g * _L, _L)]
    inr = (i16 >= base) & (i16 < base + nrows_w)
    nin = jnp.sum(inr.astype(jnp.int32))

    @pl.when(nin > 0)
    def _():
      _, keep = plsc.scan_count(i16, mask=inr)
      keep = keep & inr
      local = jnp.where(inr, i16 - base, 0)
      plsc.store_scatter(win_v, [local], g * _L + iota, mask=keep)
    return 0
  lax.fori_loop(0, _NG, scan, 0)

  def do_chunk(c, nrows, ngroups):
    def grp(j, cnt):
      w16 = win_v[pl.ds(c * _CHUNK + j * _L, _L)]
      has = w16 >= 0
      plsc.store_compressed(glist_v.at[pl.ds(cnt, _L)], w16, mask=has)
      plsc.store_compressed(plist_v.at[pl.ds(cnt, _L)], j * _L + iota,
                            mask=has)
      return cnt + jnp.sum(has.astype(jnp.int32))
    cnt = lax.fori_loop(0, ngroups, grp, jnp.int32(0))

    @pl.when(cnt > 0)
    def _():
      ntile = (cnt + _L - 1) // _L
      def fire(t, _):
        gi = glist_v[pl.ds(t * _L, _L)]
        gi = jnp.where(t * _L + iota < cnt, gi, iota)
        pltpu.async_copy(inp_hbm.at[gi], cbuf_v.at[pl.ds(t * _L, _L)],
                         gsem).wait()
        return 0
      lax.fori_loop(0, ntile, fire, 0)

      def place(j, _):
        p = plist_v[pl.ds(j, _L)][0]
        def col(l, _):
          chunk_v[p, pl.ds(l * _L, _L)] = cbuf_v[j, pl.ds(l * _L, _L)]
          return 0
        lax.fori_loop(0, _C // _L, col, 0)
        return 0
      lax.fori_loop(0, cnt, place, 0)

    pltpu.sync_copy(chunk_v.at[pl.ds(0, nrows)],
                    out_hbm.at[pl.ds(base + c * _CHUNK, nrows)])

    @pl.when(cnt > 0)
    def _():
      def rez(j, _):
        p = plist_v[pl.ds(j, _L)][0]
        def col(l, _):
          chunk_v[p, pl.ds(l * _L, _L)] = jnp.zeros((_L,), jnp.float32)
          return 0
        lax.fori_loop(0, _C // _L, col, 0)
        return 0
      lax.fori_loop(0, cnt, rez, 0)

  def full_chunk(c, _):
    do_chunk(c, _CHUNK, _CHUNK // _L)
    return 0
  nfull_dyn = jnp.where(wid == _NW - 1, _NFULL_LAST, _NFULL)
  lax.fori_loop(0, nfull_dyn, full_chunk, 0)

  @pl.when(wid < _NW - 1)
  def _():
    do_chunk(jnp.int32(_NFULL), _REM, (_REM + _L - 1) // _L)

  @pl.when(wid == _NW - 1)
  def _():
    do_chunk(jnp.int32(_NFULL_LAST), _REM_LAST, (_REM_LAST + _L - 1) // _L)


def kernel(inputs, indices):
  idx32 = indices.astype(jnp.int32)
  mesh = plsc.VectorSubcoreMesh(core_axis_name="c", subcore_axis_name="s")
  fn = pl.kernel(
      _sc_body,
      out_type=jax.ShapeDtypeStruct((_B, _C), jnp.float32),
      mesh=mesh,
      scratch_types=[
          pltpu.VMEM((_N,), jnp.int32),
          pltpu.VMEM((_WPAD,), jnp.int32),
          pltpu.VMEM((_CHUNK + _L,), jnp.int32),
          pltpu.VMEM((_CHUNK + _L,), jnp.int32),
          pltpu.VMEM((_CHUNK, _C), jnp.float32),
          pltpu.VMEM((_CHUNK, _C), jnp.float32),
          pltpu.SemaphoreType.DMA,
      ],
      compiler_params=pltpu.CompilerParams(needs_layout_passes=False),
  )
  return fn(inputs, idx32)


# speedup vs baseline: 1.6642x; 1.6642x over previous
import jax
import jax.numpy as jnp
from jax import lax
from jax.experimental import pallas as pl
from jax.experimental.pallas import tpu as pltpu
from jax.experimental.pallas import tpu_sc as plsc

_B = 100000
_C = 128
_N = 16384
_L = 16
_NC = 2
_NS = 16
_NW = _NC * _NS
_RPW = 3128
_RLAST = _B - (_NW - 1) * _RPW
_NG = _N // _L
_ZCH = 128
_NFULL = _RPW // _ZCH
_REM = _RPW - _NFULL * _ZCH
_NFULL_LAST = _RLAST // _ZCH
_REM_LAST = _RLAST - _NFULL_LAST * _ZCH
_WPAD = ((_RPW + _L - 1) // _L) * _L
_PIPE = 4


def _sc_body(inp_hbm, idx_hbm, out_hbm,
             idx_v, win_v, wlist_v, rlist_v, zbuf_v, cbuf_v,
             zsem, gsem, ssem):
  wid = lax.axis_index("s") * _NC + lax.axis_index("c")
  base = wid * _RPW
  nrows_w = jnp.where(wid == _NW - 1, _RLAST, _RPW)
  nfull_dyn = jnp.where(wid == _NW - 1, _NFULL_LAST, _NFULL)
  iota = lax.iota(jnp.int32, _L)

  def zb(r, _):
    def zc(l, _):
      zbuf_v[r, pl.ds(l * _L, _L)] = jnp.zeros((_L,), jnp.float32)
      return 0
    lax.fori_loop(0, _C // _L, zc, 0)
    return 0
  lax.fori_loop(0, _ZCH, zb, 0)

  def zfire(k, _):
    pltpu.async_copy(zbuf_v, out_hbm.at[pl.ds(base + k * _ZCH, _ZCH)], zsem)
    return 0
  lax.fori_loop(0, nfull_dyn, zfire, 0)

  @pl.when(wid < _NW - 1)
  def _():
    pltpu.async_copy(zbuf_v.at[pl.ds(0, _REM)],
                     out_hbm.at[pl.ds(base + _NFULL * _ZCH, _REM)], zsem)

  @pl.when(wid == _NW - 1)
  def _():
    pltpu.async_copy(zbuf_v.at[pl.ds(0, _REM_LAST)],
                     out_hbm.at[pl.ds(base + _NFULL_LAST * _ZCH, _REM_LAST)],
                     zsem)

  pltpu.sync_copy(idx_hbm, idx_v)

  def init_w(g, _):
    win_v[pl.ds(g * _L, _L)] = jnp.full((_L,), -1, jnp.int32)
    return 0
  lax.fori_loop(0, _WPAD // _L, init_w, 0)

  def scan(g, _):
    i16 = idx_v[pl.ds(g * _L, _L)]
    inr = (i16 >= base) & (i16 < base + nrows_w)
    nin = jnp.sum(inr.astype(jnp.int32))

    @pl.when(nin > 0)
    def _():
      _, keep = plsc.scan_count(i16, mask=inr)
      keep = keep & inr
      local = jnp.where(inr, i16 - base, 0)
      plsc.store_scatter(win_v, [local], g * _L + iota, mask=keep)
    return 0
  lax.fori_loop(0, _NG, scan, 0)

  def lg(g, cnt):
    w16 = win_v[pl.ds(g * _L, _L)]
    has = w16 >= 0
    plsc.store_compressed(wlist_v.at[pl.ds(cnt, _L)], w16, mask=has)
    plsc.store_compressed(rlist_v.at[pl.ds(cnt, _L)], base + g * _L + iota,
                          mask=has)
    return cnt + jnp.sum(has.astype(jnp.int32))
  cnt = lax.fori_loop(0, _WPAD // _L, lg, jnp.int32(0))

  def zdrain(k, _):
    pltpu.make_async_copy(zbuf_v, out_hbm.at[pl.ds(base, _ZCH)], zsem).wait()
    return 0
  lax.fori_loop(0, nfull_dyn, zdrain, 0)

  @pl.when(wid < _NW - 1)
  def _():
    pltpu.make_async_copy(zbuf_v.at[pl.ds(0, _REM)],
                          out_hbm.at[pl.ds(base, _REM)], zsem).wait()

  @pl.when(wid == _NW - 1)
  def _():
    pltpu.make_async_copy(zbuf_v.at[pl.ds(0, _REM_LAST)],
                          out_hbm.at[pl.ds(base, _REM_LAST)], zsem).wait()

  @pl.when(cnt > 0)
  def _():
    w0 = wlist_v[pl.ds(0, _L)][0]
    r0 = rlist_v[pl.ds(0, _L)][0]
    nblk = (cnt + _L - 1) // _L
    nstep = (nblk + _PIPE - 1) // _PIPE

    def gdrain(q, _):
      pltpu.make_async_copy(inp_hbm.at[iota], cbuf_v.at[pl.ds(0, _L)],
                            gsem).wait()
      return 0

    def sdrain(q, _):
      pltpu.make_async_copy(cbuf_v.at[pl.ds(0, _L)], out_hbm.at[iota],
                            ssem).wait()
      return 0

    def step(s, _):
      @pl.when(s > 0)
      def _():
        lax.fori_loop(0, _PIPE, sdrain, 0)

      def gfire(q, _):
        blk = s * _PIPE + q
        @pl.when(blk < nblk)
        def _():
          gi = wlist_v[pl.ds(blk * _L, _L)]
          gi = jnp.where(blk * _L + iota < cnt, gi, w0)
          pltpu.async_copy(inp_hbm.at[gi], cbuf_v.at[pl.ds(q * _L, _L)],
                           gsem)
        return 0
      lax.fori_loop(0, _PIPE, gfire, 0)

      ng = jnp.minimum(_PIPE, nblk - s * _PIPE)
      lax.fori_loop(0, ng, gdrain, 0)

      def sfire(q, _):
        blk = s * _PIPE + q
        @pl.when(blk < nblk)
        def _():
          ri = rlist_v[pl.ds(blk * _L, _L)]
          ri = jnp.where(blk * _L + iota < cnt, ri, r0)
          pltpu.async_copy(cbuf_v.at[pl.ds(q * _L, _L)], out_hbm.at[ri],
                           ssem)
        return 0
      lax.fori_loop(0, _PIPE, sfire, 0)
      return 0
    lax.fori_loop(0, nstep, step, 0)

    nlast = nblk - _PIPE * (nstep - 1)
    lax.fori_loop(0, nlast, sdrain, 0)


def kernel(inputs, indices):
  idx32 = indices.astype(jnp.int32)
  mesh = plsc.VectorSubcoreMesh(core_axis_name="c", subcore_axis_name="s")
  fn = pl.kernel(
      _sc_body,
      out_type=jax.ShapeDtypeStruct((_B, _C), jnp.float32),
      mesh=mesh,
      scratch_types=[
          pltpu.VMEM((_N,), jnp.int32),
          pltpu.VMEM((_WPAD,), jnp.int32),
          pltpu.VMEM((_WPAD + _L,), jnp.int32),
          pltpu.VMEM((_WPAD + _L,), jnp.int32),
          pltpu.VMEM((_ZCH, _C), jnp.float32),
          pltpu.VMEM((_PIPE * _L, _C), jnp.float32),
          pltpu.SemaphoreType.DMA,
          pltpu.SemaphoreType.DMA,
          pltpu.SemaphoreType.DMA,
      ],
      compiler_params=pltpu.CompilerParams(needs_layout_passes=False),
  )
  return fn(inputs, idx32)


# speedup vs baseline: 1.9404x; 1.1660x over previous
import jax
import jax.numpy as jnp
from jax import lax
from jax.experimental import pallas as pl
from jax.experimental.pallas import tpu as pltpu
from jax.experimental.pallas import tpu_sc as plsc

_B = 100000
_C = 128
_N = 16384
_L = 16
_NC = 2
_NS = 16
_NW = _NC * _NS
_RPW = 3128
_RLAST = _B - (_NW - 1) * _RPW
_NG = _N // _L
_ZCH = 128
_NFULL = _RPW // _ZCH
_REM = _RPW - _NFULL * _ZCH
_NFULL_LAST = _RLAST // _ZCH
_REM_LAST = _RLAST - _NFULL_LAST * _ZCH
_WPAD = ((_RPW + _L - 1) // _L) * _L
_BLK = 128
_LPAD = _WPAD + _BLK + 2 * _L


def _sc_body(inp_hbm, idx_hbm, out_hbm,
             idx_v, win_v, wlist_v, rlist_v, zbuf_v,
             cbufA_v, cbufB_v, rstgA_v, rstgB_v,
             zsem, gsem, ssem):
  wid = lax.axis_index("s") * _NC + lax.axis_index("c")
  base = wid * _RPW
  nrows_w = jnp.where(wid == _NW - 1, _RLAST, _RPW)
  nfull_dyn = jnp.where(wid == _NW - 1, _NFULL_LAST, _NFULL)
  iota = lax.iota(jnp.int32, _L)

  pltpu.sync_copy(idx_hbm, idx_v)

  def zb(r, _):
    def zc(l, _):
      zbuf_v[r, pl.ds(l * _L, _L)] = jnp.zeros((_L,), jnp.float32)
      return 0
    lax.fori_loop(0, _C // _L, zc, 0)
    return 0
  lax.fori_loop(0, _ZCH, zb, 0)

  def zfire(k, _):
    pltpu.async_copy(zbuf_v, out_hbm.at[pl.ds(base + k * _ZCH, _ZCH)], zsem)
    return 0
  lax.fori_loop(0, nfull_dyn, zfire, 0)

  @pl.when(wid < _NW - 1)
  def _():
    pltpu.async_copy(zbuf_v.at[pl.ds(0, _REM)],
                     out_hbm.at[pl.ds(base + _NFULL * _ZCH, _REM)], zsem)

  @pl.when(wid == _NW - 1)
  def _():
    pltpu.async_copy(zbuf_v.at[pl.ds(0, _REM_LAST)],
                     out_hbm.at[pl.ds(base + _NFULL_LAST * _ZCH, _REM_LAST)],
                     zsem)

  def init_w(g, _):
    win_v[pl.ds(g * _L, _L)] = jnp.full((_L,), -1, jnp.int32)
    return 0
  lax.fori_loop(0, _WPAD // _L, init_w, 0)

  def scan(g, _):
    i16 = idx_v[pl.ds(g * _L, _L)]
    inr = (i16 >= base) & (i16 < base + nrows_w)
    nin = jnp.sum(inr.astype(jnp.int32))

    @pl.when(nin > 0)
    def _():
      _, keep = plsc.scan_count(i16, mask=inr)
      keep = keep & inr
      local = jnp.where(inr, i16 - base, 0)
      plsc.store_scatter(win_v, [local], g * _L + iota, mask=keep)
    return 0
  lax.fori_loop(0, _NG, scan, 0)

  def lg(g, cnt):
    w16 = win_v[pl.ds(g * _L, _L)]
    has = w16 >= 0
    plsc.store_compressed(wlist_v.at[pl.ds(cnt, _L)], w16, mask=has)
    plsc.store_compressed(rlist_v.at[pl.ds(cnt, _L)], base + g * _L + iota,
                          mask=has)
    return cnt + jnp.sum(has.astype(jnp.int32))
  cnt = lax.fori_loop(0, _WPAD // _L, lg, jnp.int32(0))

  def zdrain(k, _):
    pltpu.make_async_copy(zbuf_v, out_hbm.at[pl.ds(base, _ZCH)], zsem).wait()
    return 0
  lax.fori_loop(0, nfull_dyn, zdrain, 0)

  @pl.when(wid < _NW - 1)
  def _():
    pltpu.make_async_copy(zbuf_v.at[pl.ds(0, _REM)],
                          out_hbm.at[pl.ds(base, _REM)], zsem).wait()

  @pl.when(wid == _NW - 1)
  def _():
    pltpu.make_async_copy(zbuf_v.at[pl.ds(0, _REM_LAST)],
                          out_hbm.at[pl.ds(base, _REM_LAST)], zsem).wait()

  @pl.when(cnt > 0)
  def _():
    w0 = wlist_v[pl.ds(0, _L)][0]
    r0 = rlist_v[pl.ds(0, _L)][0]

    def san(j, _):
      off = cnt + j * _L
      wlist_v[pl.ds(off, _L)] = jnp.full((_L,), w0, jnp.int32)
      rlist_v[pl.ds(off, _L)] = jnp.full((_L,), r0, jnp.int32)
      return 0
    lax.fori_loop(0, _BLK // _L + 1, san, 0)

    nblk = (cnt + _BLK - 1) // _BLK

    def gfire(b, cbuf):
      pltpu.async_copy(inp_hbm.at[wlist_v.at[pl.ds(b * _BLK, _BLK)]],
                       cbuf, gsem)

    def gdrain():
      pltpu.make_async_copy(inp_hbm.at[wlist_v.at[pl.ds(0, _BLK)]],
                            cbufA_v, gsem).wait()

    def sdrain(q, _):
      pltpu.make_async_copy(cbufA_v, out_hbm.at[rstgA_v], ssem).wait()
      return 0

    def prep_and_sfire(b, cbuf, rstg):
      def cp(j, _):
        rstg[pl.ds(j * _L, _L)] = rlist_v[pl.ds(b * _BLK + j * _L, _L)]
        return 0
      lax.fori_loop(0, _BLK // _L, cp, 0)
      pltpu.async_copy(cbuf, out_hbm.at[rstg], ssem)

    gfire(jnp.int32(0), cbufA_v)

    def blk(b, _):
      def work(cbuf, rstg):
        gdrain()
        @pl.when(b >= 1)
        def _():
          sdrain(0, 0)
        @pl.when(b + 1 < nblk)
        def _():
          other = cbufB_v if cbuf is cbufA_v else cbufA_v
          gfire(b + 1, other)
        prep_and_sfire(b, cbuf, rstg)

      @pl.when(b % 2 == 0)
      def _():
        work(cbufA_v, rstgA_v)

      @pl.when(b % 2 == 1)
      def _():
        work(cbufB_v, rstgB_v)
      return 0
    lax.fori_loop(0, nblk, blk, 0)

    sdrain(0, 0)


def kernel(inputs, indices):
  idx32 = indices.astype(jnp.int32)
  mesh = plsc.VectorSubcoreMesh(core_axis_name="c", subcore_axis_name="s")
  fn = pl.kernel(
      _sc_body,
      out_type=jax.ShapeDtypeStruct((_B, _C), jnp.float32),
      mesh=mesh,
      scratch_types=[
          pltpu.VMEM((_N,), jnp.int32),
          pltpu.VMEM((_WPAD,), jnp.int32),
          pltpu.VMEM((_LPAD,), jnp.int32),
          pltpu.VMEM((_LPAD,), jnp.int32),
          pltpu.VMEM((_ZCH, _C), jnp.float32),
          pltpu.VMEM((_BLK, _C), jnp.float32),
          pltpu.VMEM((_BLK, _C), jnp.float32),
          pltpu.VMEM((_BLK,), jnp.int32),
          pltpu.VMEM((_BLK,), jnp.int32),
          pltpu.SemaphoreType.DMA,
          pltpu.SemaphoreType.DMA,
          pltpu.SemaphoreType.DMA,
      ],
      compiler_params=pltpu.CompilerParams(needs_layout_passes=False),
  )
  return fn(inputs, idx32)


# speedup vs baseline: 2.0340x; 1.0482x over previous
import jax
import jax.numpy as jnp
from jax import lax
from jax.experimental import pallas as pl
from jax.experimental.pallas import tpu as pltpu
from jax.experimental.pallas import tpu_sc as plsc

_B = 100000
_C = 128
_N = 16384
_L = 16
_NC = 2
_NS = 16
_NW = _NC * _NS
_RPW = 3128
_RLAST = _B - (_NW - 1) * _RPW
_NG = _N // _L
_ZCH = 128
_NFULL = _RPW // _ZCH
_REM = _RPW - _NFULL * _ZCH
_NFULL_LAST = _RLAST // _ZCH
_REM_LAST = _RLAST - _NFULL_LAST * _ZCH
_WPAD = ((_RPW + _L - 1) // _L) * _L
_BLK = 128
_LPAD = _WPAD + _BLK + 2 * _L


def _sc_body(inp_hbm, idx_hbm, out_hbm,
             idx_v, win_v, wlist_v, rlist_v, zbuf_v,
             cbufA_v, cbufB_v, rstgA_v, rstgB_v,
             zsem, gsem, ssem):
  wid = lax.axis_index("s") * _NC + lax.axis_index("c")
  base = wid * _RPW
  nrows_w = jnp.where(wid == _NW - 1, _RLAST, _RPW)
  nfull_dyn = jnp.where(wid == _NW - 1, _NFULL_LAST, _NFULL)
  iota = lax.iota(jnp.int32, _L)

  pltpu.sync_copy(idx_hbm, idx_v)

  def zb(r, _):
    def zc(l, _):
      zbuf_v[r, pl.ds(l * _L, _L)] = jnp.zeros((_L,), jnp.float32)
      return 0
    lax.fori_loop(0, _C // _L, zc, 0)
    return 0
  lax.fori_loop(0, _ZCH, zb, 0)

  def zfire(k, _):
    pltpu.async_copy(zbuf_v, out_hbm.at[pl.ds(base + k * _ZCH, _ZCH)], zsem)
    return 0
  lax.fori_loop(0, nfull_dyn, zfire, 0)

  @pl.when(wid < _NW - 1)
  def _():
    pltpu.async_copy(zbuf_v.at[pl.ds(0, _REM)],
                     out_hbm.at[pl.ds(base + _NFULL * _ZCH, _REM)], zsem)

  @pl.when(wid == _NW - 1)
  def _():
    pltpu.async_copy(zbuf_v.at[pl.ds(0, _REM_LAST)],
                     out_hbm.at[pl.ds(base + _NFULL_LAST * _ZCH, _REM_LAST)],
                     zsem)

  def init_w(g, _):
    win_v[pl.ds(g * _L, _L)] = jnp.full((_L,), -1, jnp.int32)
    return 0
  lax.fori_loop(0, _WPAD // _L, init_w, 0)

  def scan(g, _):
    i16 = idx_v[pl.ds(g * _L, _L)]
    inr = (i16 >= base) & (i16 < base + nrows_w)
    nin = plsc.all_reduce_population_count(inr)[0]

    @pl.when(nin > 0)
    def _():
      _, keep = plsc.scan_count(i16, mask=inr)
      keep = keep & inr
      local = jnp.where(inr, i16 - base, 0)
      plsc.store_scatter(win_v, [local], g * _L + iota, mask=keep)
    return 0
  lax.fori_loop(0, _NG, scan, 0)

  def lg(g, cnt):
    w16 = win_v[pl.ds(g * _L, _L)]
    has = w16 >= 0
    plsc.store_compressed(wlist_v.at[pl.ds(cnt, _L)], w16, mask=has)
    plsc.store_compressed(rlist_v.at[pl.ds(cnt, _L)], base + g * _L + iota,
                          mask=has)
    return cnt + plsc.all_reduce_population_count(has)[0]
  cnt = lax.fori_loop(0, _WPAD // _L, lg, jnp.int32(0))

  def zdrain(k, _):
    pltpu.make_async_copy(zbuf_v, out_hbm.at[pl.ds(base, _ZCH)], zsem).wait()
    return 0
  lax.fori_loop(0, nfull_dyn, zdrain, 0)

  @pl.when(wid < _NW - 1)
  def _():
    pltpu.make_async_copy(zbuf_v.at[pl.ds(0, _REM)],
                          out_hbm.at[pl.ds(base, _REM)], zsem).wait()

  @pl.when(wid == _NW - 1)
  def _():
    pltpu.make_async_copy(zbuf_v.at[pl.ds(0, _REM_LAST)],
                          out_hbm.at[pl.ds(base, _REM_LAST)], zsem).wait()

  @pl.when(cnt > 0)
  def _():
    w0 = wlist_v[pl.ds(0, _L)][0]
    r0 = rlist_v[pl.ds(0, _L)][0]

    def san(j, _):
      off = cnt + j * _L
      wlist_v[pl.ds(off, _L)] = jnp.full((_L,), w0, jnp.int32)
      rlist_v[pl.ds(off, _L)] = jnp.full((_L,), r0, jnp.int32)
      return 0
    lax.fori_loop(0, _BLK // _L + 1, san, 0)

    nblk = (cnt + _BLK - 1) // _BLK

    def gfire(b, cbuf):
      pltpu.async_copy(inp_hbm.at[wlist_v.at[pl.ds(b * _BLK, _BLK)]],
                       cbuf, gsem)

    def gdrain():
      pltpu.make_async_copy(inp_hbm.at[wlist_v.at[pl.ds(0, _BLK)]],
                            cbufA_v, gsem).wait()

    def sdrain(q, _):
      pltpu.make_async_copy(cbufA_v, out_hbm.at[rstgA_v], ssem).wait()
      return 0

    def prep_and_sfire(b, cbuf, rstg):
      def cp(j, _):
        rstg[pl.ds(j * _L, _L)] = rlist_v[pl.ds(b * _BLK + j * _L, _L)]
        return 0
      lax.fori_loop(0, _BLK // _L, cp, 0)
      pltpu.async_copy(cbuf, out_hbm.at[rstg], ssem)

    gfire(jnp.int32(0), cbufA_v)

    def blk(b, _):
      def work(cbuf, rstg):
        gdrain()
        @pl.when(b >= 1)
        def _():
          sdrain(0, 0)
        @pl.when(b + 1 < nblk)
        def _():
          other = cbufB_v if cbuf is cbufA_v else cbufA_v
          gfire(b + 1, other)
        prep_and_sfire(b, cbuf, rstg)

      @pl.when(b % 2 == 0)
      def _():
        work(cbufA_v, rstgA_v)

      @pl.when(b % 2 == 1)
      def _():
        work(cbufB_v, rstgB_v)
      return 0
    lax.fori_loop(0, nblk, blk, 0)

    sdrain(0, 0)


def kernel(inputs, indices):
  idx32 = indices.astype(jnp.int32)
  mesh = plsc.VectorSubcoreMesh(core_axis_name="c", subcore_axis_name="s")
  fn = pl.kernel(
      _sc_body,
      out_type=jax.ShapeDtypeStruct((_B, _C), jnp.float32),
      mesh=mesh,
      scratch_types=[
          pltpu.VMEM((_N,), jnp.int32),
          pltpu.VMEM((_WPAD,), jnp.int32),
          pltpu.VMEM((_LPAD,), jnp.int32),
          pltpu.VMEM((_LPAD,), jnp.int32),
          pltpu.VMEM((_ZCH, _C), jnp.float32),
          pltpu.VMEM((_BLK, _C), jnp.float32),
          pltpu.VMEM((_BLK, _C), jnp.float32),
          pltpu.VMEM((_BLK,), jnp.int32),
          pltpu.VMEM((_BLK,), jnp.int32),
          pltpu.SemaphoreType.DMA,
          pltpu.SemaphoreType.DMA,
          pltpu.SemaphoreType.DMA,
      ],
      compiler_params=pltpu.CompilerParams(needs_layout_passes=False),
  )
  return fn(inputs, idx32)
